# Initial kernel scaffold; baseline (speedup 1.0000x reference)
import jax
import jax.numpy as jnp
from jax.experimental import pallas as pl


def kernel(X, enc_W1, enc_b1, enc_g1, enc_be1, enc_W2, enc_b2, dec_W1, dec_b1, dec_g1, dec_be1, dec_W2, dec_b2, codebook):
    raise NotImplementedError("write your pallas kernel here")


# speedup vs baseline: 1.2126x; 1.2126x over previous
import jax
import jax.numpy as jnp
from jax.experimental import pallas as pl
from jax.experimental.pallas import tpu as pltpu

N, D = 16384, 512
H, C = 128, 32
K = 1024
BN = 512
NB = N // BN
EPS = 1e-5
INV_N = 1.0 / N


def _mm(a, b):
    return jnp.dot(a.astype(jnp.bfloat16), b.astype(jnp.bfloat16),
                   preferred_element_type=jnp.float32)


def _body(X_ref, W1_ref, b1_ref, g1_ref, be1_ref, W2_ref, b2_ref,
          dW1_ref, db1_ref, dg1_ref, dbe1_ref, dW2_ref, db2_ref,
          cb_ref, cbT_ref, b2row_ref,
          topics_ref, loss_ref,
          h1_scr, h2_scr, s1, v1, s2, v2, zl, se):
    p = pl.program_id(0)
    b = pl.program_id(1)
    rows = pl.ds(b * BN, BN)

    @pl.when(p == 0)
    def _p0():
        h = _mm(X_ref[...], W1_ref[...]) + b1_ref[...]
        h1_scr[rows, :] = h
        blk = jnp.sum(h, axis=0, keepdims=True)

        @pl.when(b == 0)
        def _():
            s1[...] = blk

        @pl.when(b != 0)
        def _():
            s1[...] += blk

    @pl.when(p == 1)
    def _p1():
        mu = s1[...] * INV_N
        d = h1_scr[rows, :] - mu
        blk = jnp.sum(d * d, axis=0, keepdims=True)

        @pl.when(b == 0)
        def _():
            v1[...] = blk

        @pl.when(b != 0)
        def _():
            v1[...] += blk

    @pl.when(p == 2)
    def _p2():
        mu = s1[...] * INV_N
        sd = jnp.sqrt(v1[...] * INV_N + EPS)
        t = (h1_scr[rows, :] - mu) / sd * g1_ref[...] + be1_ref[...]
        r = jnp.maximum(t, 0.0)
        z = _mm(r, W2_ref[...]) + b2_ref[...]
        a2 = jnp.sum(z * z, axis=1, keepdims=True)
        ab = _mm(z, cbT_ref[...])
        dist = (a2 - 2.0 * ab) + b2row_ref[...]
        mn = jnp.min(dist, axis=1, keepdims=True)
        iota = jax.lax.broadcasted_iota(jnp.int32, (BN, K), 1)
        am = jnp.min(jnp.where(dist == mn, iota, K), axis=1, keepdims=True)
        topics_ref[rows, :] = am
        blk_zl = jnp.sum(mn, axis=0, keepdims=True)

        @pl.when(b == 0)
        def _():
            zl[...] = blk_zl

        @pl.when(b != 0)
        def _():
            zl[...] += blk_zl

        oh = (iota == am).astype(jnp.bfloat16)
        q = jnp.dot(oh, cb_ref[...].astype(jnp.bfloat16),
                    preferred_element_type=jnp.float32)
        h2 = _mm(q, dW1_ref[...]) + db1_ref[...]
        h2_scr[rows, :] = h2
        blk_s2 = jnp.sum(h2, axis=0, keepdims=True)

        @pl.when(b == 0)
        def _():
            s2[...] = blk_s2

        @pl.when(b != 0)
        def _():
            s2[...] += blk_s2

    @pl.when(p == 3)
    def _p3():
        mu = s2[...] * INV_N
        d = h2_scr[rows, :] - mu
        blk = jnp.sum(d * d, axis=0, keepdims=True)

        @pl.when(b == 0)
        def _():
            v2[...] = blk

        @pl.when(b != 0)
        def _():
            v2[...] += blk

    @pl.when(p == 4)
    def _p4():
        mu = s2[...] * INV_N
        sd = jnp.sqrt(v2[...] * INV_N + EPS)
        t = (h2_scr[rows, :] - mu) / sd * dg1_ref[...] + dbe1_ref[...]
        r = jnp.maximum(t, 0.0)
        xr = _mm(r, dW2_ref[...]) + db2_ref[...]
        d = xr - X_ref[...]
        blk = jnp.sum(jnp.sum(d * d, axis=1, keepdims=True), axis=0,
                      keepdims=True)

        @pl.when(b == 0)
        def _():
            se[...] = blk

        @pl.when(b != 0)
        def _():
            se[...] += blk

        @pl.when(b == NB - 1)
        def _():
            loss_ref[...] = (zl[...] + zl[...]) + jnp.sqrt(se[...])


def _x_index(p, b):
    use = jnp.logical_or(p == 0, p == 4)
    return (jax.lax.select(use, b, 0), 0)


def _const(p, b):
    return (0, 0)


def kernel(X, enc_W1, enc_b1, enc_g1, enc_be1, enc_W2, enc_b2,
           dec_W1, dec_b1, dec_g1, dec_be1, dec_W2, dec_b2, codebook):
    f32 = jnp.float32
    b2row = jnp.sum(codebook * codebook, axis=1).reshape(1, K)
    cbT = codebook.T

    row = lambda v: v.reshape(1, -1)
    in_specs = [
        pl.BlockSpec((BN, D), _x_index),
        pl.BlockSpec((D, H), _const),
        pl.BlockSpec((1, H), _const),
        pl.BlockSpec((1, H), _const),
        pl.BlockSpec((1, H), _const),
        pl.BlockSpec((H, C), _const),
        pl.BlockSpec((1, C), _const),
        pl.BlockSpec((C, H), _const),
        pl.BlockSpec((1, H), _const),
        pl.BlockSpec((1, H), _const),
        pl.BlockSpec((1, H), _const),
        pl.BlockSpec((H, D), _const),
        pl.BlockSpec((1, D), _const),
        pl.BlockSpec((K, C), _const),
        pl.BlockSpec((C, K), _const),
        pl.BlockSpec((1, K), _const),
    ]
    out_specs = [
        pl.BlockSpec((N, 1), _const),
        pl.BlockSpec((1, 1), _const),
    ]
    topics2d, loss2d = pl.pallas_call(
        _body,
        grid=(5, NB),
        in_specs=in_specs,
        out_specs=out_specs,
        out_shape=[
            jax.ShapeDtypeStruct((N, 1), jnp.int32),
            jax.ShapeDtypeStruct((1, 1), f32),
        ],
        scratch_shapes=[
            pltpu.VMEM((N, H), f32),
            pltpu.VMEM((N, H), f32),
            pltpu.VMEM((1, H), f32),
            pltpu.VMEM((1, H), f32),
            pltpu.VMEM((1, H), f32),
            pltpu.VMEM((1, H), f32),
            pltpu.VMEM((1, 1), f32),
            pltpu.VMEM((1, 1), f32),
        ],
    )(X, enc_W1, row(enc_b1), row(enc_g1), row(enc_be1), enc_W2,
      row(enc_b2), dec_W1, row(dec_b1), row(dec_g1), row(dec_be1), dec_W2,
      row(dec_b2), codebook, cbT, b2row)
    return topics2d.reshape(N), loss2d[0, 0]
